# baseline (speedup 1.0000x reference)
import functools

import jax
import jax.numpy as jnp
from jax import lax
from jax.experimental import pallas as pl
from jax.experimental.pallas import tpu as pltpu
from jax.experimental.pallas import tpu_sc as plsc

TASK_NUMS = 1000
BLOCK_ROWS = 100000
D = 64
BATCH = 16384
NC = 2
NS = 16
NW = NC * NS
TASK_PER_W = BATCH // NW
GATHER_CHUNK = 128
N_GATHER = TASK_PER_W // GATHER_CHUNK

BLOCK_CHUNKS_PER_W = 6
BLOCK_CHUNK = 528
BLOCK_LAST_START = BLOCK_ROWS - BLOCK_CHUNK


def _make_kernel():
    mesh = plsc.VectorSubcoreMesh(core_axis_name="c", subcore_axis_name="s")

    @functools.partial(
        pl.kernel,
        mesh=mesh,
        out_type=jax.ShapeDtypeStruct((BATCH + BLOCK_ROWS, D), jnp.float32),
        scratch_types=[
            pltpu.VMEM((N_GATHER, GATHER_CHUNK), jnp.int32),
            pltpu.VMEM((TASK_PER_W, D), jnp.float32),
            pltpu.VMEM((BLOCK_CHUNK, D), jnp.float32),
            pltpu.VMEM((BLOCK_CHUNK, D), jnp.float32),
            pltpu.SemaphoreType.DMA,
            pltpu.SemaphoreType.DMA,
            pltpu.SemaphoreType.DMA,
        ],
        compiler_params=pltpu.CompilerParams(use_tc_tiling_on_sc=False),
    )
    def k(idx_hbm, task_w_hbm, block_w_hbm, out_hbm,
          idx_v, rows_v, blk_a, blk_b, gsem, rsem, wsem):
        wid = lax.axis_index("s") * NC + lax.axis_index("c")
        tbase = wid * TASK_PER_W

        def chunk_start(j):
            return pl.multiple_of(
                jnp.minimum((wid * BLOCK_CHUNKS_PER_W + j) * BLOCK_CHUNK,
                            BLOCK_LAST_START), 8)

        bufs = (blk_a, blk_b)

        reads = [pltpu.async_copy(
            block_w_hbm.at[pl.ds(chunk_start(0), BLOCK_CHUNK)], blk_a, rsem)]

        pltpu.sync_copy(idx_hbm.at[wid], idx_v)
        gathers = [
            pltpu.async_copy(
                task_w_hbm.at[idx_v.at[j]],
                rows_v.at[pl.ds(j * GATHER_CHUNK, GATHER_CHUNK)],
                gsem)
            for j in range(N_GATHER)
        ]

        writes = [None] * BLOCK_CHUNKS_PER_W
        for j in range(BLOCK_CHUNKS_PER_W):
            if j + 1 < BLOCK_CHUNKS_PER_W:
                if j - 1 >= 0:
                    writes[j - 1].wait()
                reads.append(pltpu.async_copy(
                    block_w_hbm.at[pl.ds(chunk_start(j + 1), BLOCK_CHUNK)],
                    bufs[(j + 1) % 2], rsem))
            reads[j].wait()
            writes[j] = pltpu.async_copy(
                bufs[j % 2],
                out_hbm.at[pl.ds(BATCH + chunk_start(j), BLOCK_CHUNK)],
                wsem)

        for g in gathers:
            g.wait()
        pltpu.sync_copy(rows_v, out_hbm.at[pl.ds(tbase, TASK_PER_W)])

        writes[-2].wait()
        writes[-1].wait()

    return k


_sc_kernel = _make_kernel()


def kernel(task_ids, task_embs_weight, block_emb_weight, block_emb_input):
    del block_emb_input
    idx = task_ids.reshape(NW, N_GATHER, GATHER_CHUNK)
    return _sc_kernel(idx, task_embs_weight, block_emb_weight)
